# baseline (speedup 1.0000x reference)
import jax
import jax.numpy as jnp
from jax import lax
from jax.experimental import pallas as pl
from jax.experimental.pallas import tpu as pltpu
from jax.experimental.pallas import tpu_sc as plsc

IN_F = 16384
OUT_F = 4096
NNZ = 131072
B = 256
EPS = 1e-5
NEG_SLOPE = 0.01

NC = 2
NS = 16
NW = NC * NS
L = 16

SB = 4
NPASS = B // (SB * NW)
E = 4096
NCHUNK = NNZ // E
NGRP = E // L
UNROLL = 4


def _sc_body(feat_hbm, packed_hbm, y_hbm,
             fs0, fs1, ac0, ac1, ac2, ac3, ac4, ac5, ac6, ac7,
             eb0, eb1, sem, fsem, osem):
    c = lax.axis_index("c")
    s = lax.axis_index("s")
    wid = s * NC + c
    fs = (fs0, fs1)
    acc = ((ac0, ac1, ac2, ac3), (ac4, ac5, ac6, ac7))

    def start(k, buf):
        pltpu.async_copy(packed_hbm.at[pl.ds(k * 3 * E, 3 * E)], buf, sem)

    def wait(buf):
        pltpu.make_async_copy(packed_hbm.at[pl.ds(0, 3 * E)], buf, sem).wait()

    for p in range(NPASS):
        ac = acc[p]
        b0 = wid * SB + p * (SB * NW)
        start(0, eb0)
        start(1, eb1)
        j0 = b0 // 2
        fdmas = [pltpu.async_copy(feat_hbm.at[pl.ds((j0 + q) * IN_F, IN_F)],
                                  fs[q], fsem) for q in range(SB // 2)]
        z = jnp.zeros((L,), jnp.float32)

        def zcol(j, cc):
            for b in range(SB):
                for u in range(2):
                    ac[b][pl.ds((j * 2 + u) * L, L)] = z
            return cc

        lax.fori_loop(0, OUT_F // (2 * L), zcol, 0)
        for d in fdmas:
            d.wait()

        def compute(eb):
            @plsc.parallel_loop(0, NGRP, step=1, unroll=UNROLL)
            def grp(g):
                ii = eb[pl.ds(g * L, L)]
                oo = eb[pl.ds(E + g * L, L)]
                ww = plsc.bitcast(eb[pl.ds(2 * E + g * L, L)], jnp.float32)
                for q in range(SB // 2):
                    pk = plsc.load_gather(fs[q], [ii])
                    va, vb = plsc.unpack(plsc.bitcast(pk, jnp.bfloat16),
                                         format=plsc.PackFormat.INTERLEAVED)
                    plsc.addupdate_scatter(ac[2 * q], [oo], va * ww)
                    plsc.addupdate_scatter(ac[2 * q + 1], [oo], vb * ww)

        def outer(t, cc):
            wait(eb0)
            compute(eb0)
            start(2 * t + 2, eb0)
            wait(eb1)
            compute(eb1)
            start(2 * t + 3, eb1)
            return cc

        lax.fori_loop(0, NCHUNK // 2 - 1, outer, 0)
        wait(eb0)
        compute(eb0)
        wait(eb1)
        compute(eb1)

        for b in range(SB):
            pltpu.async_copy(ac[b], y_hbm.at[pl.ds((b0 + b) * OUT_F, OUT_F)],
                             osem)

    for p in range(NPASS):
        for b in range(SB):
            pltpu.make_async_copy(acc[p][b], y_hbm.at[pl.ds(0, OUT_F)],
                                  osem).wait()


def _sc_call(packf, packed):
    mesh = plsc.VectorSubcoreMesh(core_axis_name="c", subcore_axis_name="s",
                                  num_cores=NC, num_subcores=NS)
    f = pl.kernel(
        _sc_body,
        out_type=jax.ShapeDtypeStruct((B * OUT_F,), jnp.float32),
        mesh=mesh,
        compiler_params=pltpu.CompilerParams(needs_layout_passes=False),
        scratch_types=(
            [pltpu.VMEM((IN_F,), jnp.int32) for _ in range(SB // 2)]
            + [pltpu.VMEM((OUT_F,), jnp.float32) for _ in range(NPASS * SB)]
            + [pltpu.VMEM((3 * E,), jnp.int32),
               pltpu.VMEM((3 * E,), jnp.int32),
               pltpu.SemaphoreType.DMA,
               pltpu.SemaphoreType.DMA,
               pltpu.SemaphoreType.DMA]
        ),
    )
    return f(packf, packed).reshape(B, OUT_F)


def _tc_body(y_ref, bias_ref, o_ref):
    y = y_ref[...] + bias_ref[...]
    mean = jnp.mean(y, axis=0, keepdims=True)
    d = y - mean
    var = jnp.mean(d * d, axis=0, keepdims=True)
    yn = d * lax.rsqrt(var + EPS)
    o_ref[...] = jnp.where(yn >= 0, yn, NEG_SLOPE * yn)


def _tc_call(y_lin, bias):
    bo = 512
    return pl.pallas_call(
        _tc_body,
        grid=(OUT_F // bo,),
        in_specs=[
            pl.BlockSpec((B, bo), lambda i: (0, i)),
            pl.BlockSpec((1, bo), lambda i: (0, i)),
        ],
        out_specs=pl.BlockSpec((B, bo), lambda i: (0, i)),
        out_shape=jax.ShapeDtypeStruct((B, OUT_F), jnp.float32),
    )(y_lin, bias)


@jax.jit
def kernel(features, in_idx, out_idx, weights, bias):
    in_idx = in_idx.astype(jnp.int32)
    out_idx = out_idx.astype(jnp.int32)
    w_bits = lax.bitcast_convert_type(weights, jnp.int32)
    packed = jnp.stack([in_idx.reshape(NCHUNK, E),
                        out_idx.reshape(NCHUNK, E),
                        w_bits.reshape(NCHUNK, E)], axis=1).reshape(-1)
    fb = features.astype(jnp.bfloat16).reshape(B // 2, 2, IN_F)
    packf = lax.bitcast_convert_type(jnp.moveaxis(fb, 1, 2),
                                     jnp.int32).reshape(-1)
    y_lin = _sc_call(packf, packed)
    return _tc_call(y_lin, bias.reshape(1, OUT_F))


# speedup vs baseline: 1.1735x; 1.1735x over previous
import jax
import jax.numpy as jnp
from jax import lax
from jax.experimental import pallas as pl
from jax.experimental.pallas import tpu as pltpu
from jax.experimental.pallas import tpu_sc as plsc

IN_F = 16384
OUT_F = 4096
NNZ = 131072
B = 256
EPS = 1e-5
NEG_SLOPE = 0.01

NC = 2
NS = 16
NW = NC * NS
L = 16

SB = 4
NPASS = B // (SB * NW)
E = 4096
NCHUNK = NNZ // E
NGRP = E // L
UNROLL = 4


def _sc_body(feat_hbm, packed_hbm, y_hbm,
             fs0, fs1, fs2, fs3, ac0, ac1, ac2, ac3, ac4, ac5, ac6, ac7,
             eb0, eb1, sem, fsem, osem):
    c = lax.axis_index("c")
    s = lax.axis_index("s")
    wid = s * NC + c
    fs = (fs0, fs1, fs2, fs3)
    acc = ((ac0, ac1, ac2, ac3), (ac4, ac5, ac6, ac7))

    def start(k, buf):
        pltpu.async_copy(packed_hbm.at[pl.ds(k * 3 * E, 3 * E)], buf, sem)

    def wait(buf):
        pltpu.make_async_copy(packed_hbm.at[pl.ds(0, 3 * E)], buf, sem).wait()

    for p in range(NPASS):
        ac = acc[p]
        b0 = wid * SB + p * (SB * NW)
        start(0, eb0)
        start(1, eb1)
        fdmas = [pltpu.async_copy(feat_hbm.at[pl.ds((b0 + b) * IN_F, IN_F)],
                                  fs[b], fsem) for b in range(SB)]
        z = jnp.zeros((L,), jnp.float32)

        def zcol(j, cc):
            for b in range(SB):
                for u in range(2):
                    ac[b][pl.ds((j * 2 + u) * L, L)] = z
            return cc

        lax.fori_loop(0, OUT_F // (2 * L), zcol, 0)
        for d in fdmas:
            d.wait()

        def compute(eb):
            @plsc.parallel_loop(0, NGRP, step=1, unroll=UNROLL)
            def grp(g):
                ii = eb[pl.ds(g * L, L)]
                oo = eb[pl.ds(E + g * L, L)]
                ww = plsc.bitcast(eb[pl.ds(2 * E + g * L, L)], jnp.float32)
                for b in range(SB):
                    vals = plsc.load_gather(fs[b], [ii]) * ww
                    plsc.addupdate_scatter(ac[b], [oo], vals)

        def outer(t, cc):
            wait(eb0)
            compute(eb0)
            start(2 * t + 2, eb0)
            wait(eb1)
            compute(eb1)
            start(2 * t + 3, eb1)
            return cc

        lax.fori_loop(0, NCHUNK // 2 - 1, outer, 0)
        wait(eb0)
        compute(eb0)
        wait(eb1)
        compute(eb1)

        for b in range(SB):
            pltpu.async_copy(ac[b], y_hbm.at[pl.ds((b0 + b) * OUT_F, OUT_F)],
                             osem)

    for p in range(NPASS):
        for b in range(SB):
            pltpu.make_async_copy(acc[p][b], y_hbm.at[pl.ds(0, OUT_F)],
                                  osem).wait()


def _sc_call(features, packed):
    mesh = plsc.VectorSubcoreMesh(core_axis_name="c", subcore_axis_name="s",
                                  num_cores=NC, num_subcores=NS)
    f = pl.kernel(
        _sc_body,
        out_type=jax.ShapeDtypeStruct((B * OUT_F,), jnp.float32),
        mesh=mesh,
        compiler_params=pltpu.CompilerParams(needs_layout_passes=False),
        scratch_types=(
            [pltpu.VMEM((IN_F,), jnp.float32) for _ in range(SB)]
            + [pltpu.VMEM((OUT_F,), jnp.float32) for _ in range(NPASS * SB)]
            + [pltpu.VMEM((3 * E,), jnp.int32),
               pltpu.VMEM((3 * E,), jnp.int32),
               pltpu.SemaphoreType.DMA,
               pltpu.SemaphoreType.DMA,
               pltpu.SemaphoreType.DMA]
        ),
    )
    return f(features.reshape(B * IN_F), packed).reshape(B, OUT_F)


def _tc_body(y_ref, bias_ref, o_ref):
    y = y_ref[...] + bias_ref[...]
    mean = jnp.mean(y, axis=0, keepdims=True)
    d = y - mean
    var = jnp.mean(d * d, axis=0, keepdims=True)
    yn = d * lax.rsqrt(var + EPS)
    o_ref[...] = jnp.where(yn >= 0, yn, NEG_SLOPE * yn)


def _tc_call(y_lin, bias):
    bo = 512
    return pl.pallas_call(
        _tc_body,
        grid=(OUT_F // bo,),
        in_specs=[
            pl.BlockSpec((B, bo), lambda i: (0, i)),
            pl.BlockSpec((1, bo), lambda i: (0, i)),
        ],
        out_specs=pl.BlockSpec((B, bo), lambda i: (0, i)),
        out_shape=jax.ShapeDtypeStruct((B, OUT_F), jnp.float32),
    )(y_lin, bias)


@jax.jit
def kernel(features, in_idx, out_idx, weights, bias):
    in_idx = in_idx.astype(jnp.int32)
    out_idx = out_idx.astype(jnp.int32)
    w_bits = lax.bitcast_convert_type(weights, jnp.int32)
    packed = jnp.stack([in_idx.reshape(NCHUNK, E),
                        out_idx.reshape(NCHUNK, E),
                        w_bits.reshape(NCHUNK, E)], axis=1).reshape(-1)
    y_lin = _sc_call(features, packed)
    return _tc_call(y_lin, bias.reshape(1, OUT_F))
